# Initial kernel scaffold; baseline (speedup 1.0000x reference)
import jax
import jax.numpy as jnp
from jax.experimental import pallas as pl


def kernel(hidden_states, mu, sigma, W_combine, b_combine):
    raise NotImplementedError("write your pallas kernel here")


# speedup vs baseline: 2.6387x; 2.6387x over previous
import functools

import jax
import jax.numpy as jnp
from jax.experimental import pallas as pl
from jax.experimental.pallas import tpu as pltpu

Q = 20
EPS = 1e-8
BB = 8
L = 13
K = 11


def _body(x_ref, wcls_ref, mu_ref, c_ref, w_ref, b_ref, out_ref):
    l = pl.program_id(1)
    x = x_ref[0]

    g_rows = []
    for i in range(BB):
        chunks = []
        for c in range(0, 512, 128):
            xc = x[i, c:c + 128, :]
            s2 = jnp.sum(xc * xc, axis=-1, keepdims=True)
            r = 1.0 / jnp.maximum(jnp.sqrt(s2), EPS)
            chunks.append(xc * r)
        n = jnp.concatenate(chunks, axis=0)
        qn = chunks[0][:Q, :]

        sim = jax.lax.dot_general(
            qn.astype(jnp.bfloat16), n.astype(jnp.bfloat16),
            (((1,), (1,)), ((), ())),
            preferred_element_type=jnp.float32)

        f_acc = None
        for k in range(K):
            e = sim - mu_ref[k]
            term = w_ref[l, k] * jnp.exp(c_ref[k] * e * e)
            f_acc = term if f_acc is None else f_acc + term
        g_rows.append(jnp.sum(f_acc, axis=0))

    g = jnp.stack(g_rows, axis=0)
    dmask = jax.lax.broadcasted_iota(jnp.int32, (BB, 512), 1) >= Q
    g = jnp.where(dmask, g, 0.0)
    contrib = jnp.sum(g, axis=-1, keepdims=True)

    @pl.when(l == 0)
    def _():
        out_ref[...] = contrib + b_ref[0]

    @pl.when(l != 0)
    def _():
        out_ref[...] = out_ref[...] + contrib

    @pl.when(l == L - 1)
    def _():
        cls = x[:, 0, :]
        cc = jnp.sum(cls * wcls_ref[...], axis=-1, keepdims=True)
        out_ref[...] = out_ref[...] + cc


@functools.partial(jax.jit, static_argnames=("interpret",))
def kernel(hidden_states, mu, sigma, W_combine, b_combine, interpret=False):
    B = hidden_states.shape[1]
    w = W_combine[0]
    wcls = w[:768].reshape(1, 768)
    wk = w[768:].reshape(L + 1, K)
    w_eff = jnp.concatenate([(wk[0] + wk[1])[None, :], wk[2:]], axis=0)
    c = -0.5 / (sigma * sigma)

    out = pl.pallas_call(
        _body,
        out_shape=jax.ShapeDtypeStruct((B, 1), jnp.float32),
        grid=(B // BB, L),
        in_specs=[
            pl.BlockSpec((1, BB, 512, 768), lambda bb, l: (l, bb, 0, 0)),
            pl.BlockSpec((1, 768), lambda bb, l: (0, 0)),
            pl.BlockSpec(memory_space=pltpu.SMEM),
            pl.BlockSpec(memory_space=pltpu.SMEM),
            pl.BlockSpec(memory_space=pltpu.SMEM),
            pl.BlockSpec(memory_space=pltpu.SMEM),
        ],
        out_specs=pl.BlockSpec((BB, 1), lambda bb, l: (bb, 0)),
        compiler_params=pltpu.CompilerParams(
            dimension_semantics=("parallel", "arbitrary"),
            vmem_limit_bytes=56 * 1024 * 1024,
        ),
        name="cedr_knrm",
        interpret=interpret,
    )(hidden_states, wcls, mu, c, w_eff, b_combine)
    return out


# speedup vs baseline: 2.7578x; 1.0451x over previous
import functools

import jax
import jax.numpy as jnp
from jax.experimental import pallas as pl
from jax.experimental.pallas import tpu as pltpu

Q = 20
EPS = 1e-8
BB = 16
L = 13
K = 11


def _body(x_ref, wcls_ref, mu_ref, c_ref, w_ref, b_ref, out_ref):
    l = pl.program_id(1)
    x = x_ref[0]

    g_rows = []
    for i in range(BB):
        chunks = []
        for c in range(0, 512, 128):
            xc = x[i, c:c + 128, :]
            s2 = jnp.sum(xc * xc, axis=-1, keepdims=True)
            r = 1.0 / jnp.maximum(jnp.sqrt(s2), EPS)
            chunks.append(xc * r)
        n = jnp.concatenate(chunks, axis=0)
        qn = chunks[0][:Q, :]

        sim = jax.lax.dot_general(
            qn.astype(jnp.bfloat16), n.astype(jnp.bfloat16),
            (((1,), (1,)), ((), ())),
            preferred_element_type=jnp.float32)

        f_acc = None
        for k in range(K):
            e = sim - mu_ref[k]
            term = w_ref[l, k] * jnp.exp(c_ref[k] * e * e)
            f_acc = term if f_acc is None else f_acc + term
        g_rows.append(jnp.sum(f_acc, axis=0))

    g = jnp.stack(g_rows, axis=0)
    dmask = jax.lax.broadcasted_iota(jnp.int32, (BB, 512), 1) >= Q
    g = jnp.where(dmask, g, 0.0)
    contrib = jnp.sum(g, axis=-1, keepdims=True)

    @pl.when(l == 0)
    def _():
        out_ref[...] = contrib + b_ref[0]

    @pl.when(l != 0)
    def _():
        out_ref[...] = out_ref[...] + contrib

    @pl.when(l == L - 1)
    def _():
        cls = x[:, 0, :]
        cc = jnp.sum(cls * wcls_ref[...], axis=-1, keepdims=True)
        out_ref[...] = out_ref[...] + cc


@functools.partial(jax.jit, static_argnames=("interpret",))
def kernel(hidden_states, mu, sigma, W_combine, b_combine, interpret=False):
    B = hidden_states.shape[1]
    w = W_combine[0]
    wcls = w[:768].reshape(1, 768)
    wk = w[768:].reshape(L + 1, K)
    w_eff = jnp.concatenate([(wk[0] + wk[1])[None, :], wk[2:]], axis=0)
    c = -0.5 / (sigma * sigma)

    out = pl.pallas_call(
        _body,
        out_shape=jax.ShapeDtypeStruct((B, 1), jnp.float32),
        grid=(B // BB, L),
        in_specs=[
            pl.BlockSpec((1, BB, 512, 768), lambda bb, l: (l, bb, 0, 0)),
            pl.BlockSpec((1, 768), lambda bb, l: (0, 0)),
            pl.BlockSpec(memory_space=pltpu.SMEM),
            pl.BlockSpec(memory_space=pltpu.SMEM),
            pl.BlockSpec(memory_space=pltpu.SMEM),
            pl.BlockSpec(memory_space=pltpu.SMEM),
        ],
        out_specs=pl.BlockSpec((BB, 1), lambda bb, l: (bb, 0)),
        compiler_params=pltpu.CompilerParams(
            dimension_semantics=("parallel", "arbitrary"),
            vmem_limit_bytes=56 * 1024 * 1024,
        ),
        name="cedr_knrm",
        interpret=interpret,
    )(hidden_states, wcls, mu, c, w_eff, b_combine)
    return out


# speedup vs baseline: 2.7656x; 1.0028x over previous
import jax
import jax.numpy as jnp
from jax.experimental import pallas as pl
from jax.experimental.pallas import tpu as pltpu

Q = 20
EPS = 1e-8
BB = 16
L = 13
K = 11


def _body(x_ref, wcls_ref, mu_ref, c_ref, w_ref, b_ref, out_ref):
    l = pl.program_id(1)
    x = x_ref[0]

    g_rows = []
    for i in range(BB):
        chunks = []
        for c in range(0, 512, 128):
            xc = x[i, c:c + 128, :]
            s2 = jnp.sum(xc * xc, axis=-1, keepdims=True)
            r = 1.0 / jnp.maximum(jnp.sqrt(s2), EPS)
            chunks.append(xc * r)
        n = jnp.concatenate(chunks, axis=0)
        qn = chunks[0][:Q, :]

        sim = jax.lax.dot_general(
            qn.astype(jnp.bfloat16), n.astype(jnp.bfloat16),
            (((1,), (1,)), ((), ())),
            preferred_element_type=jnp.float32)

        f_acc = None
        for k in range(K):
            e = sim - mu_ref[k]
            term = w_ref[l, k] * jnp.exp(c_ref[k] * e * e)
            f_acc = term if f_acc is None else f_acc + term
        g_rows.append(jnp.sum(f_acc, axis=0))

    g = jnp.stack(g_rows, axis=0)
    dmask = jax.lax.broadcasted_iota(jnp.int32, (BB, 512), 1) >= Q
    g = jnp.where(dmask, g, 0.0)
    contrib = jnp.sum(g, axis=-1, keepdims=True)

    @pl.when(l == 0)
    def _():
        out_ref[...] = contrib + b_ref[0]

    @pl.when(l != 0)
    def _():
        out_ref[...] = out_ref[...] + contrib

    @pl.when(l == L - 1)
    def _():
        cls = x[:, 0, :]
        cc = jnp.sum(cls * wcls_ref[...], axis=-1, keepdims=True)
        out_ref[...] = out_ref[...] + cc


def kernel(hidden_states, mu, sigma, W_combine, b_combine):
    B = hidden_states.shape[1]
    w = W_combine[0]
    wcls = w[:768].reshape(1, 768)
    wk = w[768:].reshape(L + 1, K)
    w_eff = jnp.concatenate([(wk[0] + wk[1])[None, :], wk[2:]], axis=0)
    c = -0.5 / (sigma * sigma)

    out = pl.pallas_call(
        _body,
        out_shape=jax.ShapeDtypeStruct((B, 1), jnp.float32),
        grid=(B // BB, L),
        in_specs=[
            pl.BlockSpec((1, BB, 512, 768), lambda bb, l: (l, bb, 0, 0)),
            pl.BlockSpec((1, 768), lambda bb, l: (0, 0)),
            pl.BlockSpec(memory_space=pltpu.SMEM),
            pl.BlockSpec(memory_space=pltpu.SMEM),
            pl.BlockSpec(memory_space=pltpu.SMEM),
            pl.BlockSpec(memory_space=pltpu.SMEM),
        ],
        out_specs=pl.BlockSpec((BB, 1), lambda bb, l: (bb, 0)),
        compiler_params=pltpu.CompilerParams(
            dimension_semantics=("parallel", "arbitrary"),
            vmem_limit_bytes=56 * 1024 * 1024,
        ),
        name="cedr_knrm",
    )(hidden_states, wcls, mu, c, w_eff, b_combine)
    return out
